# baseline (speedup 1.0000x reference)
import functools

import jax
import jax.numpy as jnp
from jax import lax
from jax.experimental import pallas as pl
from jax.experimental.pallas import tpu as pltpu
from jax.experimental.pallas import tpu_sc as plsc

NSUB = 16
NCORE = 2
DF = 64
LANES = 128
CJ = 4
STILE = 128


def _pad_to(n, m):
    return -(-n // m) * m


def _deg_body(np_, rows_per_w, dst_hbm, out_hbm, idx_v, ones_v, fill_v, dacc):
    c = lax.axis_index("c")
    s = lax.axis_index("s")
    rps = np_ // NSUB

    @pl.loop(0, LANES, step=16)
    def _(i):
        ones_v[pl.ds(i, 16)] = jnp.full((16,), 1.0, jnp.float32)

    @pl.loop(0, rps, step=16)
    def _(i):
        fill_v[pl.ds(i, 16)] = jnp.zeros((16,), jnp.float32)

    nslice = pl.ds(s * rps, rps)
    pltpu.sync_copy(fill_v, dacc.at[nslice])
    plsc.subcore_barrier()

    base = (c * NSUB + s) * rows_per_w

    @pl.loop(0, rows_per_w, step=8)
    def _(r):
        pltpu.sync_copy(dst_hbm.at[pl.ds(base + r, 8)], idx_v)
        for j in range(8):
            pltpu.sync_copy(ones_v, dacc.at[idx_v.at[j]], add=True)

    plsc.subcore_barrier()
    pltpu.sync_copy(dacc.at[nslice], out_hbm.at[c].at[nslice])


def _hop_body(np_, rows_per_w, y_hbm, src_hbm, dst_hbm, out_hbm,
              sidx0, didx0, sidx1, didx1, row0, row1, acc, g0, g1, t0, t1):
    c = lax.axis_index("c")
    s = lax.axis_index("s")
    rps = np_ // NSUB
    nbase = s * rps
    nslice = pl.ds(nbase, rps)
    ebase = (c * NSUB + s) * rows_per_w
    nch = rows_per_w // CJ

    def fireG(sidx, row, sem):
        for j in range(CJ):
            pltpu.async_copy(y_hbm.at[sidx.at[j]], row.at[j], sem)

    def drain(row, sem):
        for j in range(CJ):
            pltpu.make_async_copy(y_hbm.at[pl.ds(0, LANES)],
                                  row.at[j], sem).wait()

    def fireS(didx, row, sem):
        for j in range(CJ):
            pltpu.async_copy(row.at[j], acc.at[didx.at[j]], sem, add=True)

    def load_idx(k, sidx, didx):
        pltpu.sync_copy(src_hbm.at[pl.ds(ebase + k * CJ, CJ)], sidx)
        pltpu.sync_copy(dst_hbm.at[pl.ds(ebase + k * CJ, CJ)], didx)

    @pl.loop(0, STILE)
    def _(i):
        for q in range(DF // 16):
            row0[0, i, pl.ds(q * 16, 16)] = jnp.zeros((16,), jnp.float32)

    @pl.loop(0, rps, step=STILE)
    def _(t):
        pltpu.sync_copy(row0.at[0].at[pl.ds(0, STILE)],
                        acc.at[pl.ds(nbase + t, STILE)])

    plsc.subcore_barrier()

    load_idx(0, sidx0, didx0)
    fireG(sidx0, row0, g0)
    load_idx(1, sidx1, didx1)
    drain(row0, g0)
    fireG(sidx1, row1, g1)
    fireS(didx0, row0, t0)
    drain(row0, t0)
    load_idx(2, sidx0, didx0)
    drain(row1, g1)
    fireG(sidx0, row0, g0)
    fireS(didx1, row1, t1)

    @pl.loop(2, nch - 2, step=2)
    def _(k):
        drain(row1, t1)
        load_idx(k + 1, sidx1, didx1)
        drain(row0, g0)
        fireG(sidx1, row1, g1)
        fireS(didx0, row0, t0)
        drain(row0, t0)
        load_idx(k + 2, sidx0, didx0)
        drain(row1, g1)
        fireG(sidx0, row0, g0)
        fireS(didx1, row1, t1)

    drain(row1, t1)
    load_idx(nch - 1, sidx1, didx1)
    drain(row0, g0)
    fireG(sidx1, row1, g1)
    fireS(didx0, row0, t0)
    drain(row0, t0)
    drain(row1, g1)
    fireS(didx1, row1, t1)
    drain(row1, t1)

    plsc.subcore_barrier()
    pltpu.sync_copy(acc.at[nslice], out_hbm.at[c].at[nslice])


def _mm_body(x_ref, w_ref, p0_ref, p1_ref, y_ref, dinv_ref, dis_ref):
    deg = 1.0 + p0_ref[...] + p1_ref[...]
    dis = lax.rsqrt(deg)
    dinv_ref[...] = 1.0 / deg
    dis_ref[...] = dis
    xw = lax.dot_general(x_ref[...], w_ref[...], (((1,), (1,)), ((), ())),
                         preferred_element_type=jnp.float32)
    y_ref[...] = dis * xw


def _mid_body(hp_ref, y0_ref, dinv_ref, o_ref):
    o_ref[...] = dinv_ref[...] * (hp_ref[0] + hp_ref[1] + y0_ref[...])


def _fin_body(hp_ref, y1_ref, dis_ref, b_ref, o_ref):
    o_ref[...] = dis_ref[...] * (hp_ref[0] + hp_ref[1] + y1_ref[...]) \
        + b_ref[...]


@jax.jit
def kernel(x, edge_index, W, b):
    n, d_in = x.shape
    d_out = W.shape[0]
    e = edge_index.shape[1]

    np_ = _pad_to(n, NSUB * LANES)
    ep = _pad_to(e, NCORE * NSUB * CJ * LANES * 4)
    erows = ep // LANES

    x = x.astype(jnp.float32)
    src = edge_index[0].astype(jnp.int32)
    dst = edge_index[1].astype(jnp.int32)
    pad = jnp.full((ep - e,), np_ - 1, jnp.int32)
    src2 = jnp.concatenate([src, pad]).reshape(erows, LANES)
    dst2 = jnp.concatenate([dst, pad]).reshape(erows, LANES)
    x_pad = jnp.pad(x, ((0, np_ - n), (0, 0)))

    mesh = plsc.VectorSubcoreMesh(core_axis_name="c", subcore_axis_name="s")
    f32 = jnp.float32
    sc_params = pltpu.CompilerParams(use_tc_tiling_on_sc=False)
    rps = np_ // NSUB

    deg_call = pl.kernel(
        functools.partial(_deg_body, np_, erows // (NCORE * NSUB)),
        out_type=jax.ShapeDtypeStruct((NCORE, np_), f32),
        mesh=mesh,
        scratch_types=[
            pltpu.VMEM((8, LANES), jnp.int32),
            pltpu.VMEM((LANES,), f32),
            pltpu.VMEM((rps,), f32),
            pltpu.VMEM_SHARED((np_,), f32),
        ],
        compiler_params=sc_params,
    )
    hop_call = pl.kernel(
        functools.partial(_hop_body, np_, erows // (NCORE * NSUB)),
        out_type=jax.ShapeDtypeStruct((NCORE, np_, DF), f32),
        mesh=mesh,
        scratch_types=[
            pltpu.VMEM((CJ, LANES), jnp.int32),
            pltpu.VMEM((CJ, LANES), jnp.int32),
            pltpu.VMEM((CJ, LANES), jnp.int32),
            pltpu.VMEM((CJ, LANES), jnp.int32),
            pltpu.VMEM((CJ, LANES, DF), f32),
            pltpu.VMEM((CJ, LANES, DF), f32),
            pltpu.VMEM_SHARED((np_, DF), f32),
            pltpu.SemaphoreType.DMA,
            pltpu.SemaphoreType.DMA,
            pltpu.SemaphoreType.DMA,
            pltpu.SemaphoreType.DMA,
        ],
        compiler_params=sc_params,
    )

    blk = 512
    grid = (np_ // blk,)
    col1 = lambda i: (i, 0)
    mm_call = pl.pallas_call(
        _mm_body,
        grid=grid,
        in_specs=[
            pl.BlockSpec((blk, d_in), col1),
            pl.BlockSpec((d_out, d_in), lambda i: (0, 0)),
            pl.BlockSpec((blk, 1), col1),
            pl.BlockSpec((blk, 1), col1),
        ],
        out_specs=[
            pl.BlockSpec((blk, DF), col1),
            pl.BlockSpec((blk, 1), col1),
            pl.BlockSpec((blk, 1), col1),
        ],
        out_shape=[
            jax.ShapeDtypeStruct((np_, DF), f32),
            jax.ShapeDtypeStruct((np_, 1), f32),
            jax.ShapeDtypeStruct((np_, 1), f32),
        ],
    )
    mid_call = pl.pallas_call(
        _mid_body,
        grid=grid,
        in_specs=[
            pl.BlockSpec((NCORE, blk, DF), lambda i: (0, i, 0)),
            pl.BlockSpec((blk, DF), col1),
            pl.BlockSpec((blk, 1), col1),
        ],
        out_specs=pl.BlockSpec((blk, DF), col1),
        out_shape=jax.ShapeDtypeStruct((np_, DF), f32),
    )
    fin_call = pl.pallas_call(
        _fin_body,
        grid=grid,
        in_specs=[
            pl.BlockSpec((NCORE, blk, DF), lambda i: (0, i, 0)),
            pl.BlockSpec((blk, DF), col1),
            pl.BlockSpec((blk, 1), col1),
            pl.BlockSpec((1, d_out), lambda i: (0, 0)),
        ],
        out_specs=pl.BlockSpec((blk, d_out), col1),
        out_shape=jax.ShapeDtypeStruct((np_, d_out), f32),
    )

    p = deg_call(dst2)
    y0, dinv, dis = mm_call(x_pad, W, p[0][:, None], p[1][:, None])
    h1p = hop_call(y0, src2, dst2)
    y1 = mid_call(h1p, y0, dinv)
    h2p = hop_call(y1, src2, dst2)
    out = fin_call(h2p, y1, dis, b.astype(f32).reshape(1, d_out))
    return out[:n]


# speedup vs baseline: 1.8931x; 1.8931x over previous
import functools

import jax
import jax.numpy as jnp
from jax import lax
from jax.experimental import pallas as pl
from jax.experimental.pallas import tpu as pltpu
from jax.experimental.pallas import tpu_sc as plsc

NSUB = 16
NCORE = 2
HALF = 32
LANES = 128
CJ = 8
STILE = 128


def _pad_to(n, m):
    return -(-n // m) * m


def _deg_body(np_, rows_per_w, dst_hbm, out_hbm, idx_v, ones_v, fill_v, dacc):
    c = lax.axis_index("c")
    s = lax.axis_index("s")
    rps = np_ // NSUB

    @pl.loop(0, LANES, step=16)
    def _(i):
        ones_v[pl.ds(i, 16)] = jnp.full((16,), 1.0, jnp.float32)

    @pl.loop(0, rps, step=16)
    def _(i):
        fill_v[pl.ds(i, 16)] = jnp.zeros((16,), jnp.float32)

    nslice = pl.ds(s * rps, rps)
    pltpu.sync_copy(fill_v, dacc.at[nslice])
    plsc.subcore_barrier()

    base = (c * NSUB + s) * rows_per_w

    @pl.loop(0, rows_per_w, step=CJ)
    def _(r):
        pltpu.sync_copy(dst_hbm.at[pl.ds(base + r, CJ)], idx_v)
        for j in range(CJ):
            pltpu.sync_copy(ones_v, dacc.at[idx_v.at[j]], add=True)

    plsc.subcore_barrier()
    pltpu.sync_copy(dacc.at[nslice], out_hbm.at[c].at[nslice])


def _hops_body(np_, rows_per_s, y_hbm, src_hbm, dst_hbm, dinv_hbm, dis_hbm,
               b_hbm, out_hbm,
               sidx0, didx0, sidx1, didx1, row0, row1, sbuf, bvec,
               dsm, accA, accB, g0, g1, t0, t1):
    c = lax.axis_index("c")
    s = lax.axis_index("s")
    rps = np_ // NSUB
    nbase = s * rps
    nslice = pl.ds(nbase, rps)
    ebase = s * rows_per_s
    nch = rows_per_s // CJ

    def fireG(src_ref, sidx, row, sem):
        for j in range(CJ):
            pltpu.async_copy(src_ref.at[sidx.at[j]], row.at[j], sem)

    def drain(row, sem):
        for j in range(CJ):
            pltpu.make_async_copy(y_hbm.at[c].at[pl.ds(0, LANES)],
                                  row.at[j], sem).wait()

    def fireS(didx, row, acc, sem):
        for j in range(CJ):
            pltpu.async_copy(row.at[j], acc.at[didx.at[j]], sem, add=True)

    def load_idx(k, sidx, didx):
        pltpu.sync_copy(src_hbm.at[pl.ds(ebase + k * CJ, CJ)], sidx)
        pltpu.sync_copy(dst_hbm.at[pl.ds(ebase + k * CJ, CJ)], didx)

    def edge_pass(src_ref, acc):
        load_idx(0, sidx0, didx0)
        fireG(src_ref, sidx0, row0, g0)
        load_idx(1, sidx1, didx1)
        drain(row0, g0)
        fireG(src_ref, sidx1, row1, g1)
        fireS(didx0, row0, acc, t0)
        drain(row0, t0)
        load_idx(2, sidx0, didx0)
        drain(row1, g1)
        fireG(src_ref, sidx0, row0, g0)
        fireS(didx1, row1, acc, t1)
        @pl.loop(2, nch - 2, step=2)
        def _(r):
            drain(row1, t1)
            load_idx(r + 1, sidx1, didx1)
            drain(row0, g0)
            fireG(src_ref, sidx1, row1, g1)
            fireS(didx0, row0, acc, t0)
            drain(row0, t0)
            load_idx(r + 2, sidx0, didx0)
            drain(row1, g1)
            fireG(src_ref, sidx0, row0, g0)
            fireS(didx1, row1, acc, t1)

        drain(row1, t1)
        load_idx(nch - 1, sidx1, didx1)
        drain(row0, g0)
        fireG(src_ref, sidx1, row1, g1)
        fireS(didx0, row0, acc, t0)
        drain(row0, t0)
        drain(row1, g1)
        fireS(didx1, row1, acc, t1)
        drain(row1, t1)

    pltpu.sync_copy(y_hbm.at[c].at[nslice], accA.at[nslice])
    pltpu.sync_copy(b_hbm.at[c], bvec)
    plsc.subcore_barrier()
    edge_pass(y_hbm.at[c], accA)
    plsc.subcore_barrier()

    pltpu.sync_copy(dinv_hbm.at[nslice], dsm)

    @pl.loop(0, rps, step=STILE)
    def _(t):
        tslice = pl.ds(nbase + t, STILE)
        pltpu.sync_copy(accA.at[tslice], sbuf)

        @pl.loop(0, STILE, step=16)
        def _(i):
            d16 = dsm[pl.ds(t + i, 16)]
            for k in range(16):
                d = d16[k]
                sbuf[i + k, pl.ds(0, 16)] = sbuf[i + k, pl.ds(0, 16)] * d
                sbuf[i + k, pl.ds(16, 16)] = sbuf[i + k, pl.ds(16, 16)] * d

        pltpu.sync_copy(sbuf, accA.at[tslice])
        pltpu.sync_copy(sbuf, accB.at[tslice])

    plsc.subcore_barrier()

    edge_pass(accB, accA)
    plsc.subcore_barrier()

    pltpu.sync_copy(dis_hbm.at[nslice], dsm)
    b0 = bvec[pl.ds(0, 16)]
    b1 = bvec[pl.ds(16, 16)]

    @pl.loop(0, rps, step=STILE)
    def _(t):
        pltpu.sync_copy(accA.at[pl.ds(nbase + t, STILE)], sbuf)

        @pl.loop(0, STILE, step=16)
        def _(i):
            d16 = dsm[pl.ds(t + i, 16)]
            for k in range(16):
                d = d16[k]
                sbuf[i + k, pl.ds(0, 16)] = sbuf[i + k, pl.ds(0, 16)] * d + b0
                sbuf[i + k, pl.ds(16, 16)] = sbuf[i + k, pl.ds(16, 16)] * d + b1

        pltpu.sync_copy(sbuf, out_hbm.at[pl.ds(nbase + t, STILE),
                                         pl.ds(c * HALF, HALF)])


def _xw_body(x_ref, w_ref, xw_ref):
    xw_ref[...] = lax.dot_general(x_ref[...], w_ref[...],
                                  (((1,), (1,)), ((), ())),
                                  preferred_element_type=jnp.float32)


def _scale_body(xw_ref, p0_ref, p1_ref, y_ref, dinv_ref, dis_ref):
    deg = 1.0 + p0_ref[...] + p1_ref[...]
    dis = lax.rsqrt(deg)
    dinv_ref[...] = 1.0 / deg
    dis_ref[...] = dis
    y = dis * xw_ref[...]
    y_ref[0] = y[:, :HALF]
    y_ref[1] = y[:, HALF:]


@jax.jit
def kernel(x, edge_index, W, b):
    n, d_in = x.shape
    d_out = W.shape[0]
    e = edge_index.shape[1]

    np_ = _pad_to(n, NSUB * LANES)
    ep = _pad_to(e, NCORE * NSUB * CJ * LANES)
    erows = ep // LANES

    x = x.astype(jnp.float32)
    src = edge_index[0].astype(jnp.int32)
    dst = edge_index[1].astype(jnp.int32)
    pad = jnp.full((ep - e,), np_ - 1, jnp.int32)
    src2 = jnp.concatenate([src, pad]).reshape(erows, LANES)
    dst2 = jnp.concatenate([dst, pad]).reshape(erows, LANES)
    x_pad = jnp.pad(x, ((0, np_ - n), (0, 0)))

    mesh = plsc.VectorSubcoreMesh(core_axis_name="c", subcore_axis_name="s")
    f32 = jnp.float32
    sc_params = pltpu.CompilerParams(use_tc_tiling_on_sc=False)
    rps = np_ // NSUB

    deg_call = pl.kernel(
        functools.partial(_deg_body, np_, erows // (NCORE * NSUB)),
        out_type=jax.ShapeDtypeStruct((NCORE, np_), f32),
        mesh=mesh,
        scratch_types=[
            pltpu.VMEM((CJ, LANES), jnp.int32),
            pltpu.VMEM((LANES,), f32),
            pltpu.VMEM((rps,), f32),
            pltpu.VMEM_SHARED((np_,), f32),
        ],
        compiler_params=sc_params,
    )
    hops_call = pl.kernel(
        functools.partial(_hops_body, np_, erows // NSUB),
        out_type=jax.ShapeDtypeStruct((np_, NCORE * HALF), f32),
        mesh=mesh,
        scratch_types=[
            pltpu.VMEM((CJ, LANES), jnp.int32),
            pltpu.VMEM((CJ, LANES), jnp.int32),
            pltpu.VMEM((CJ, LANES), jnp.int32),
            pltpu.VMEM((CJ, LANES), jnp.int32),
            pltpu.VMEM((CJ, LANES, HALF), f32),
            pltpu.VMEM((CJ, LANES, HALF), f32),
            pltpu.VMEM((STILE, HALF), f32),
            pltpu.VMEM((HALF,), f32),
            pltpu.VMEM((rps,), f32),
            pltpu.VMEM_SHARED((np_, HALF), f32),
            pltpu.VMEM_SHARED((np_, HALF), f32),
            pltpu.SemaphoreType.DMA,
            pltpu.SemaphoreType.DMA,
            pltpu.SemaphoreType.DMA,
            pltpu.SemaphoreType.DMA,
        ],
        compiler_params=sc_params,
    )

    blk = 512
    grid = (np_ // blk,)
    col1 = lambda i: (i, 0)
    xw_call = pl.pallas_call(
        _xw_body,
        grid=grid,
        in_specs=[
            pl.BlockSpec((blk, d_in), col1),
            pl.BlockSpec((d_out, d_in), lambda i: (0, 0)),
        ],
        out_specs=pl.BlockSpec((blk, d_out), col1),
        out_shape=jax.ShapeDtypeStruct((np_, d_out), f32),
    )
    scale_call = pl.pallas_call(
        _scale_body,
        grid=grid,
        in_specs=[
            pl.BlockSpec((blk, d_out), col1),
            pl.BlockSpec((blk, 1), col1),
            pl.BlockSpec((blk, 1), col1),
        ],
        out_specs=[
            pl.BlockSpec((NCORE, blk, HALF), lambda i: (0, i, 0)),
            pl.BlockSpec((blk, 1), col1),
            pl.BlockSpec((blk, 1), col1),
        ],
        out_shape=[
            jax.ShapeDtypeStruct((NCORE, np_, HALF), f32),
            jax.ShapeDtypeStruct((np_, 1), f32),
            jax.ShapeDtypeStruct((np_, 1), f32),
        ],
    )

    xw = xw_call(x_pad, W)
    p = deg_call(dst2)
    y0, dinv, dis = scale_call(xw, p[0][:, None], p[1][:, None])
    b2 = b.astype(f32).reshape(NCORE, HALF)
    out = hops_call(y0, src2, dst2, dinv.reshape(np_), dis.reshape(np_), b2)
    return out[:n]


# speedup vs baseline: 1.8984x; 1.0028x over previous
import functools

import jax
import jax.numpy as jnp
from jax import lax
from jax.experimental import pallas as pl
from jax.experimental.pallas import tpu as pltpu
from jax.experimental.pallas import tpu_sc as plsc

NSUB = 16
NCORE = 2
HALF = 32
LANES = 128
CJ = 8
STILE = 128


def _pad_to(n, m):
    return -(-n // m) * m


def _deg_body(np_, rows_per_w, dst_hbm, out_hbm, idx_v, ones_v, fill_v, dacc):
    c = lax.axis_index("c")
    s = lax.axis_index("s")
    rps = np_ // NSUB

    @pl.loop(0, LANES, step=16)
    def _(i):
        ones_v[pl.ds(i, 16)] = jnp.full((16,), 1.0, jnp.float32)

    @pl.loop(0, rps, step=16)
    def _(i):
        fill_v[pl.ds(i, 16)] = jnp.zeros((16,), jnp.float32)

    nslice = pl.ds(s * rps, rps)
    pltpu.sync_copy(fill_v, dacc.at[nslice])
    plsc.subcore_barrier()

    base = (c * NSUB + s) * rows_per_w

    @pl.loop(0, rows_per_w, step=CJ)
    def _(r):
        pltpu.sync_copy(dst_hbm.at[pl.ds(base + r, CJ)], idx_v)
        for j in range(CJ):
            pltpu.sync_copy(ones_v, dacc.at[idx_v.at[j]], add=True)

    plsc.subcore_barrier()
    pltpu.sync_copy(dacc.at[nslice], out_hbm.at[c].at[nslice])


def _hops_body(np_, rows_per_s, y_hbm, src_hbm, dst_hbm, dinv_hbm, dis_hbm,
               b_hbm, out_hbm,
               sidx0, didx0, sidx1, didx1, row0, row1, sbuf, bvec,
               dsm, accA, accB, g0, g1, t0, t1):
    c = lax.axis_index("c")
    s = lax.axis_index("s")
    rps = np_ // NSUB
    nbase = s * rps
    nslice = pl.ds(nbase, rps)
    ebase = s * rows_per_s
    nch = rows_per_s // CJ

    def fireG(src_ref, sidx, row, sem):
        for j in range(CJ):
            pltpu.async_copy(src_ref.at[sidx.at[j]], row.at[j], sem)

    def drain(row, sem):
        for j in range(CJ):
            pltpu.make_async_copy(y_hbm.at[c].at[pl.ds(0, LANES)],
                                  row.at[j], sem).wait()

    def fireS(didx, row, acc, sem):
        for j in range(CJ):
            pltpu.async_copy(row.at[j], acc.at[didx.at[j]], sem, add=True)

    def load_idx(k, sidx, didx):
        pltpu.sync_copy(src_hbm.at[pl.ds(ebase + k * CJ, CJ)], sidx)
        pltpu.sync_copy(dst_hbm.at[pl.ds(ebase + k * CJ, CJ)], didx)

    def edge_pass(src_ref, acc):
        load_idx(0, sidx0, didx0)
        fireG(src_ref, sidx0, row0, g0)
        load_idx(1, sidx1, didx1)
        drain(row0, g0)
        fireG(src_ref, sidx1, row1, g1)
        fireS(didx0, row0, acc, t0)
        drain(row0, t0)
        load_idx(2, sidx0, didx0)
        drain(row1, g1)
        fireG(src_ref, sidx0, row0, g0)
        fireS(didx1, row1, acc, t1)
        @pl.loop(2, nch - 2, step=2)
        def _(r):
            drain(row1, t1)
            load_idx(r + 1, sidx1, didx1)
            drain(row0, g0)
            fireG(src_ref, sidx1, row1, g1)
            fireS(didx0, row0, acc, t0)
            drain(row0, t0)
            load_idx(r + 2, sidx0, didx0)
            drain(row1, g1)
            fireG(src_ref, sidx0, row0, g0)
            fireS(didx1, row1, acc, t1)

        drain(row1, t1)
        load_idx(nch - 1, sidx1, didx1)
        drain(row0, g0)
        fireG(src_ref, sidx1, row1, g1)
        fireS(didx0, row0, acc, t0)
        drain(row0, t0)
        drain(row1, g1)
        fireS(didx1, row1, acc, t1)
        drain(row1, t1)

    pltpu.sync_copy(y_hbm.at[c].at[nslice], accA.at[nslice])
    pltpu.sync_copy(b_hbm.at[c], bvec)
    plsc.subcore_barrier()
    edge_pass(y_hbm.at[c], accA)
    plsc.subcore_barrier()

    pltpu.sync_copy(dinv_hbm.at[nslice], dsm)

    @pl.loop(0, rps, step=STILE)
    def _(t):
        tslice = pl.ds(nbase + t, STILE)
        pltpu.sync_copy(accA.at[tslice], sbuf)

        @pl.loop(0, STILE, step=16)
        def _(i):
            d16 = dsm[pl.ds(t + i, 16)]
            for k in range(16):
                d = d16[k]
                sbuf[i + k, pl.ds(0, 16)] = sbuf[i + k, pl.ds(0, 16)] * d
                sbuf[i + k, pl.ds(16, 16)] = sbuf[i + k, pl.ds(16, 16)] * d

        pltpu.sync_copy(sbuf, accA.at[tslice])
        pltpu.sync_copy(sbuf, accB.at[tslice])

    plsc.subcore_barrier()

    edge_pass(accB, accA)
    plsc.subcore_barrier()

    pltpu.sync_copy(dis_hbm.at[nslice], dsm)
    b0 = bvec[pl.ds(0, 16)]
    b1 = bvec[pl.ds(16, 16)]

    @pl.loop(0, rps, step=STILE)
    def _(t):
        pltpu.sync_copy(accA.at[pl.ds(nbase + t, STILE)], sbuf)

        @pl.loop(0, STILE, step=16)
        def _(i):
            d16 = dsm[pl.ds(t + i, 16)]
            for k in range(16):
                d = d16[k]
                sbuf[i + k, pl.ds(0, 16)] = sbuf[i + k, pl.ds(0, 16)] * d + b0
                sbuf[i + k, pl.ds(16, 16)] = sbuf[i + k, pl.ds(16, 16)] * d + b1

        pltpu.sync_copy(sbuf, out_hbm.at[pl.ds(nbase + t, STILE),
                                         pl.ds(c * HALF, HALF)])


def _mm_body(x_ref, w_ref, p0_ref, p1_ref, y_ref, dinv_ref, dis_ref):
    deg = 1.0 + p0_ref[...] + p1_ref[...]
    dis = lax.rsqrt(deg)
    dinv_ref[...] = 1.0 / deg
    dis_ref[...] = dis
    xw = lax.dot_general(x_ref[...], w_ref[...], (((1,), (1,)), ((), ())),
                         preferred_element_type=jnp.float32)
    y = dis * xw
    y_ref[0] = y[:, :HALF]
    y_ref[1] = y[:, HALF:]


@jax.jit
def kernel(x, edge_index, W, b):
    n, d_in = x.shape
    d_out = W.shape[0]
    e = edge_index.shape[1]

    np_ = _pad_to(n, NSUB * LANES)
    ep = _pad_to(e, NCORE * NSUB * CJ * LANES)
    erows = ep // LANES

    x = x.astype(jnp.float32)
    src = edge_index[0].astype(jnp.int32)
    dst = edge_index[1].astype(jnp.int32)
    pad = jnp.full((ep - e,), np_ - 1, jnp.int32)
    src2 = jnp.concatenate([src, pad]).reshape(erows, LANES)
    dst2 = jnp.concatenate([dst, pad]).reshape(erows, LANES)
    x_pad = jnp.pad(x, ((0, np_ - n), (0, 0)))

    mesh = plsc.VectorSubcoreMesh(core_axis_name="c", subcore_axis_name="s")
    f32 = jnp.float32
    sc_params = pltpu.CompilerParams(use_tc_tiling_on_sc=False)
    rps = np_ // NSUB

    deg_call = pl.kernel(
        functools.partial(_deg_body, np_, erows // (NCORE * NSUB)),
        out_type=jax.ShapeDtypeStruct((NCORE, np_), f32),
        mesh=mesh,
        scratch_types=[
            pltpu.VMEM((CJ, LANES), jnp.int32),
            pltpu.VMEM((LANES,), f32),
            pltpu.VMEM((rps,), f32),
            pltpu.VMEM_SHARED((np_,), f32),
        ],
        compiler_params=sc_params,
    )
    hops_call = pl.kernel(
        functools.partial(_hops_body, np_, erows // NSUB),
        out_type=jax.ShapeDtypeStruct((np_, NCORE * HALF), f32),
        mesh=mesh,
        scratch_types=[
            pltpu.VMEM((CJ, LANES), jnp.int32),
            pltpu.VMEM((CJ, LANES), jnp.int32),
            pltpu.VMEM((CJ, LANES), jnp.int32),
            pltpu.VMEM((CJ, LANES), jnp.int32),
            pltpu.VMEM((CJ, LANES, HALF), f32),
            pltpu.VMEM((CJ, LANES, HALF), f32),
            pltpu.VMEM((STILE, HALF), f32),
            pltpu.VMEM((HALF,), f32),
            pltpu.VMEM((rps,), f32),
            pltpu.VMEM_SHARED((np_, HALF), f32),
            pltpu.VMEM_SHARED((np_, HALF), f32),
            pltpu.SemaphoreType.DMA,
            pltpu.SemaphoreType.DMA,
            pltpu.SemaphoreType.DMA,
            pltpu.SemaphoreType.DMA,
        ],
        compiler_params=sc_params,
    )

    blk = 512
    grid = (np_ // blk,)
    col1 = lambda i: (i, 0)
    mm_call = pl.pallas_call(
        _mm_body,
        grid=grid,
        in_specs=[
            pl.BlockSpec((blk, d_in), col1),
            pl.BlockSpec((d_out, d_in), lambda i: (0, 0)),
            pl.BlockSpec((blk, 1), col1),
            pl.BlockSpec((blk, 1), col1),
        ],
        out_specs=[
            pl.BlockSpec((NCORE, blk, HALF), lambda i: (0, i, 0)),
            pl.BlockSpec((blk, 1), col1),
            pl.BlockSpec((blk, 1), col1),
        ],
        out_shape=[
            jax.ShapeDtypeStruct((NCORE, np_, HALF), f32),
            jax.ShapeDtypeStruct((np_, 1), f32),
            jax.ShapeDtypeStruct((np_, 1), f32),
        ],
    )

    p = deg_call(dst2)
    y0, dinv, dis = mm_call(x_pad, W, p[0][:, None], p[1][:, None])
    b2 = b.astype(f32).reshape(NCORE, HALF)
    out = hops_call(y0, src2, dst2, dinv.reshape(np_), dis.reshape(np_), b2)
    return out[:n]


# speedup vs baseline: 1.9575x; 1.0311x over previous
import functools

import jax
import jax.numpy as jnp
from jax import lax
from jax.experimental import pallas as pl
from jax.experimental.pallas import tpu as pltpu
from jax.experimental.pallas import tpu_sc as plsc

NSUB = 16
NCORE = 2
HALF = 32
LANES = 128
CJ = 8
STILE = 128


def _pad_to(n, m):
    return -(-n // m) * m


def _deg_body(np_, rows_per_w, dst_hbm, out_hbm, idx0, idx1, ones_v, fill_v,
              dacc, t0, t1):
    c = lax.axis_index("c")
    s = lax.axis_index("s")
    rps = np_ // NSUB

    @pl.loop(0, LANES, step=16)
    def _(i):
        ones_v[pl.ds(i, 16)] = jnp.full((16,), 1.0, jnp.float32)

    @pl.loop(0, rps, step=16)
    def _(i):
        fill_v[pl.ds(i, 16)] = jnp.zeros((16,), jnp.float32)

    nslice = pl.ds(s * rps, rps)
    pltpu.sync_copy(fill_v, dacc.at[nslice])
    plsc.subcore_barrier()

    base = (c * NSUB + s) * rows_per_w
    nch = rows_per_w // CJ

    def load_idx(k, idx):
        pltpu.sync_copy(dst_hbm.at[pl.ds(base + k * CJ, CJ)], idx)

    def fireS(idx, sem):
        for j in range(CJ):
            pltpu.async_copy(ones_v, dacc.at[idx.at[j]], sem, add=True)

    def drainS(idx, sem):
        for j in range(CJ):
            pltpu.make_async_copy(dst_hbm.at[pl.ds(0, 1)].at[0],
                                  idx.at[0].at[pl.ds(0, LANES)], sem).wait()

    load_idx(0, idx0)
    fireS(idx0, t0)
    load_idx(1, idx1)
    fireS(idx1, t1)

    @pl.loop(2, nch, step=2)
    def _(k):
        drainS(idx0, t0)
        load_idx(k, idx0)
        fireS(idx0, t0)
        drainS(idx1, t1)
        load_idx(k + 1, idx1)
        fireS(idx1, t1)

    drainS(idx0, t0)
    drainS(idx1, t1)
    plsc.subcore_barrier()
    pltpu.sync_copy(dacc.at[nslice], out_hbm.at[c].at[nslice])


def _hops_body(np_, rows_per_s, y_hbm, src_hbm, dst_hbm, dinv_hbm, dis_hbm,
               b_hbm, out_hbm,
               sidx0, didx0, sidx1, didx1, row0, row1, sbuf, bvec,
               dsm, accA, accB, g0, g1, t0, t1):
    c = lax.axis_index("c")
    s = lax.axis_index("s")
    rps = np_ // NSUB
    nbase = s * rps
    nslice = pl.ds(nbase, rps)
    ebase = s * rows_per_s
    nch = rows_per_s // CJ

    def fireG(src_ref, sidx, row, sem):
        for j in range(CJ):
            pltpu.async_copy(src_ref.at[sidx.at[j]],
                             row.at[pl.ds(j * LANES, LANES)], sem)

    def drain(row, sem):
        pltpu.make_async_copy(y_hbm.at[c].at[pl.ds(0, CJ * LANES)],
                              row, sem).wait()

    def fireS(didx, row, acc, sem):
        for j in range(CJ):
            pltpu.async_copy(row.at[pl.ds(j * LANES, LANES)],
                             acc.at[didx.at[j]], sem, add=True)

    def load_idx(k, sidx, didx):
        pltpu.sync_copy(src_hbm.at[pl.ds(ebase + k * CJ, CJ)], sidx)
        pltpu.sync_copy(dst_hbm.at[pl.ds(ebase + k * CJ, CJ)], didx)

    def edge_pass(src_ref, acc):
        load_idx(0, sidx0, didx0)
        fireG(src_ref, sidx0, row0, g0)
        load_idx(1, sidx1, didx1)
        drain(row0, g0)
        fireG(src_ref, sidx1, row1, g1)
        fireS(didx0, row0, acc, t0)
        drain(row0, t0)
        load_idx(2, sidx0, didx0)
        drain(row1, g1)
        fireG(src_ref, sidx0, row0, g0)
        fireS(didx1, row1, acc, t1)
        @pl.loop(2, nch - 2, step=2)
        def _(r):
            drain(row1, t1)
            load_idx(r + 1, sidx1, didx1)
            drain(row0, g0)
            fireG(src_ref, sidx1, row1, g1)
            fireS(didx0, row0, acc, t0)
            drain(row0, t0)
            load_idx(r + 2, sidx0, didx0)
            drain(row1, g1)
            fireG(src_ref, sidx0, row0, g0)
            fireS(didx1, row1, acc, t1)

        drain(row1, t1)
        load_idx(nch - 1, sidx1, didx1)
        drain(row0, g0)
        fireG(src_ref, sidx1, row1, g1)
        fireS(didx0, row0, acc, t0)
        drain(row0, t0)
        drain(row1, g1)
        fireS(didx1, row1, acc, t1)
        drain(row1, t1)

    pltpu.sync_copy(y_hbm.at[c].at[nslice], accA.at[nslice])
    pltpu.sync_copy(b_hbm.at[c], bvec)
    plsc.subcore_barrier()
    edge_pass(y_hbm.at[c], accA)
    plsc.subcore_barrier()

    pltpu.sync_copy(dinv_hbm.at[nslice], dsm)

    @pl.loop(0, rps, step=STILE)
    def _(t):
        tslice = pl.ds(nbase + t, STILE)
        pltpu.sync_copy(accA.at[tslice], sbuf)

        @pl.loop(0, STILE, step=16)
        def _(i):
            d16 = dsm[pl.ds(t + i, 16)]
            for k in range(16):
                d = d16[k]
                sbuf[i + k, pl.ds(0, 16)] = sbuf[i + k, pl.ds(0, 16)] * d
                sbuf[i + k, pl.ds(16, 16)] = sbuf[i + k, pl.ds(16, 16)] * d

        pltpu.sync_copy(sbuf, accA.at[tslice])
        pltpu.sync_copy(sbuf, accB.at[tslice])

    plsc.subcore_barrier()

    edge_pass(accB, accA)
    plsc.subcore_barrier()

    pltpu.sync_copy(dis_hbm.at[nslice], dsm)
    b0 = bvec[pl.ds(0, 16)]
    b1 = bvec[pl.ds(16, 16)]

    @pl.loop(0, rps, step=STILE)
    def _(t):
        pltpu.sync_copy(accA.at[pl.ds(nbase + t, STILE)], sbuf)

        @pl.loop(0, STILE, step=16)
        def _(i):
            d16 = dsm[pl.ds(t + i, 16)]
            for k in range(16):
                d = d16[k]
                sbuf[i + k, pl.ds(0, 16)] = sbuf[i + k, pl.ds(0, 16)] * d + b0
                sbuf[i + k, pl.ds(16, 16)] = sbuf[i + k, pl.ds(16, 16)] * d + b1

        pltpu.sync_copy(sbuf, out_hbm.at[pl.ds(nbase + t, STILE),
                                         pl.ds(c * HALF, HALF)])


def _mm_body(x_ref, w_ref, p_ref, y_ref, dinv_ref, dis_ref):
    deg = 1.0 + p_ref[:, 0:1] + p_ref[:, 1:2]
    dis = lax.rsqrt(deg)
    dinv_ref[...] = 1.0 / deg
    dis_ref[...] = dis
    xw = lax.dot_general(x_ref[...], w_ref[...], (((1,), (1,)), ((), ())),
                         preferred_element_type=jnp.float32)
    y = dis * xw
    y_ref[0] = y[:, :HALF]
    y_ref[1] = y[:, HALF:]


@jax.jit
def kernel(x, edge_index, W, b):
    n, d_in = x.shape
    d_out = W.shape[0]
    e = edge_index.shape[1]

    np_ = _pad_to(n, NSUB * LANES)
    ep = _pad_to(e, NCORE * NSUB * CJ * LANES)
    erows = ep // LANES

    x = x.astype(jnp.float32)
    src = edge_index[0].astype(jnp.int32)
    dst = edge_index[1].astype(jnp.int32)
    pad = jnp.full((ep - e,), np_ - 1, jnp.int32)
    src2 = jnp.concatenate([src, pad]).reshape(erows, LANES)
    dst2 = jnp.concatenate([dst, pad]).reshape(erows, LANES)
    x_pad = jnp.pad(x, ((0, np_ - n), (0, 0)))

    mesh = plsc.VectorSubcoreMesh(core_axis_name="c", subcore_axis_name="s")
    f32 = jnp.float32
    sc_params = pltpu.CompilerParams(use_tc_tiling_on_sc=False)
    rps = np_ // NSUB

    deg_call = pl.kernel(
        functools.partial(_deg_body, np_, erows // (NCORE * NSUB)),
        out_type=jax.ShapeDtypeStruct((NCORE, np_), f32),
        mesh=mesh,
        scratch_types=[
            pltpu.VMEM((CJ, LANES), jnp.int32),
            pltpu.VMEM((CJ, LANES), jnp.int32),
            pltpu.VMEM((LANES,), f32),
            pltpu.VMEM((rps,), f32),
            pltpu.VMEM_SHARED((np_,), f32),
            pltpu.SemaphoreType.DMA,
            pltpu.SemaphoreType.DMA,
        ],
        compiler_params=sc_params,
    )
    hops_call = pl.kernel(
        functools.partial(_hops_body, np_, erows // NSUB),
        out_type=jax.ShapeDtypeStruct((np_, NCORE * HALF), f32),
        mesh=mesh,
        scratch_types=[
            pltpu.VMEM((CJ, LANES), jnp.int32),
            pltpu.VMEM((CJ, LANES), jnp.int32),
            pltpu.VMEM((CJ, LANES), jnp.int32),
            pltpu.VMEM((CJ, LANES), jnp.int32),
            pltpu.VMEM((CJ * LANES, HALF), f32),
            pltpu.VMEM((CJ * LANES, HALF), f32),
            pltpu.VMEM((STILE, HALF), f32),
            pltpu.VMEM((HALF,), f32),
            pltpu.VMEM((rps,), f32),
            pltpu.VMEM_SHARED((np_, HALF), f32),
            pltpu.VMEM_SHARED((np_, HALF), f32),
            pltpu.SemaphoreType.DMA,
            pltpu.SemaphoreType.DMA,
            pltpu.SemaphoreType.DMA,
            pltpu.SemaphoreType.DMA,
        ],
        compiler_params=sc_params,
    )

    blk = 512
    grid = (np_ // blk,)
    mm_call = pl.pallas_call(
        _mm_body,
        grid=grid,
        in_specs=[
            pl.BlockSpec((blk, d_in), lambda i: (i, 0)),
            pl.BlockSpec((d_out, d_in), lambda i: (0, 0)),
            pl.BlockSpec((blk, NCORE), lambda i: (i, 0)),
        ],
        out_specs=[
            pl.BlockSpec((NCORE, blk, HALF), lambda i: (0, i, 0)),
            pl.BlockSpec((blk, 1), lambda i: (i, 0)),
            pl.BlockSpec((blk, 1), lambda i: (i, 0)),
        ],
        out_shape=[
            jax.ShapeDtypeStruct((NCORE, np_, HALF), f32),
            jax.ShapeDtypeStruct((np_, 1), f32),
            jax.ShapeDtypeStruct((np_, 1), f32),
        ],
    )

    p = deg_call(dst2)
    y0, dinv, dis = mm_call(x_pad, W, p.T)
    b2 = b.astype(f32).reshape(NCORE, HALF)
    out = hops_call(y0, src2, dst2, dinv.reshape(np_), dis.reshape(np_), b2)
    return out[:n]


# speedup vs baseline: 2.0098x; 1.0267x over previous
import functools

import jax
import jax.numpy as jnp
from jax import lax
from jax.experimental import pallas as pl
from jax.experimental.pallas import tpu as pltpu
from jax.experimental.pallas import tpu_sc as plsc

NSUB = 16
NCORE = 2
HALF = 32
LANES = 128
CJ = 8
STILE = 128


def _pad_to(n, m):
    return -(-n // m) * m


def _deg_body(np_, rows_per_w, dst_hbm, out_hbm, idx0, idx1, ones_v, fill_v,
              dacc, t0, t1):
    c = lax.axis_index("c")
    s = lax.axis_index("s")
    rps = np_ // NSUB

    @pl.loop(0, LANES, step=16)
    def _(i):
        ones_v[pl.ds(i, 16)] = jnp.full((16,), 1.0, jnp.float32)

    @pl.loop(0, rps, step=16)
    def _(i):
        fill_v[pl.ds(i, 16)] = jnp.zeros((16,), jnp.float32)

    nslice = pl.ds(s * rps, rps)
    pltpu.sync_copy(fill_v, dacc.at[nslice])
    plsc.subcore_barrier()

    base = (c * NSUB + s) * rows_per_w
    nch = rows_per_w // CJ

    def load_idx(k, idx):
        pltpu.sync_copy(dst_hbm.at[pl.ds(base + k * CJ, CJ)], idx)

    def fireS(idx, sem):
        for j in range(CJ):
            pltpu.async_copy(ones_v, dacc.at[idx.at[j]], sem, add=True)

    def drainS(idx, sem):
        for j in range(CJ):
            pltpu.make_async_copy(dst_hbm.at[pl.ds(0, 1)].at[0],
                                  idx.at[0].at[pl.ds(0, LANES)], sem).wait()

    load_idx(0, idx0)
    fireS(idx0, t0)
    load_idx(1, idx1)
    fireS(idx1, t1)

    @pl.loop(2, nch, step=2)
    def _(k):
        drainS(idx0, t0)
        load_idx(k, idx0)
        fireS(idx0, t0)
        drainS(idx1, t1)
        load_idx(k + 1, idx1)
        fireS(idx1, t1)

    drainS(idx0, t0)
    drainS(idx1, t1)
    plsc.subcore_barrier()
    pltpu.sync_copy(dacc.at[nslice], out_hbm.at[c].at[nslice])


def _hops_body(np_, rows_per_s, y_hbm, e_hbm, dinv_hbm, dis_hbm,
               b_hbm, out_hbm,
               eidx0, eidx1, row0, row1, sbuf, bvec,
               dsm, accA, accB, g0, g1, t0, t1):
    c = lax.axis_index("c")
    s = lax.axis_index("s")
    rps = np_ // NSUB
    nbase = s * rps
    nslice = pl.ds(nbase, rps)
    ebase = s * rows_per_s
    nch = rows_per_s // CJ

    def fireG(src_ref, eidx, row, sem):
        for j in range(CJ):
            pltpu.async_copy(src_ref.at[eidx.at[j, 0]],
                             row.at[pl.ds(j * LANES, LANES)], sem)

    def drain(row, sem):
        pltpu.make_async_copy(y_hbm.at[c].at[pl.ds(0, CJ * LANES)],
                              row, sem).wait()

    def fireS(eidx, row, acc, sem):
        for j in range(CJ):
            pltpu.async_copy(row.at[pl.ds(j * LANES, LANES)],
                             acc.at[eidx.at[j, 1]], sem, add=True)

    def load_idx(k, eidx):
        pltpu.sync_copy(e_hbm.at[pl.ds(ebase + k * CJ, CJ)], eidx)

    def edge_pass(src_ref, acc):
        load_idx(0, eidx0)
        fireG(src_ref, eidx0, row0, g0)
        load_idx(1, eidx1)
        drain(row0, g0)
        fireG(src_ref, eidx1, row1, g1)
        fireS(eidx0, row0, acc, t0)
        drain(row0, t0)
        load_idx(2, eidx0)
        drain(row1, g1)
        fireG(src_ref, eidx0, row0, g0)
        fireS(eidx1, row1, acc, t1)
        @pl.loop(2, nch - 2, step=2)
        def _(r):
            drain(row1, t1)
            load_idx(r + 1, eidx1)
            drain(row0, g0)
            fireG(src_ref, eidx1, row1, g1)
            fireS(eidx0, row0, acc, t0)
            drain(row0, t0)
            load_idx(r + 2, eidx0)
            drain(row1, g1)
            fireG(src_ref, eidx0, row0, g0)
            fireS(eidx1, row1, acc, t1)

        drain(row1, t1)
        load_idx(nch - 1, eidx1)
        drain(row0, g0)
        fireG(src_ref, eidx1, row1, g1)
        fireS(eidx0, row0, acc, t0)
        drain(row0, t0)
        drain(row1, g1)
        fireS(eidx1, row1, acc, t1)
        drain(row1, t1)

    pltpu.sync_copy(y_hbm.at[c].at[nslice], accA.at[nslice])
    pltpu.sync_copy(b_hbm.at[c], bvec)
    plsc.subcore_barrier()
    edge_pass(y_hbm.at[c], accA)
    plsc.subcore_barrier()

    pltpu.sync_copy(dinv_hbm.at[nslice], dsm)

    @pl.loop(0, rps, step=STILE)
    def _(t):
        tslice = pl.ds(nbase + t, STILE)
        pltpu.sync_copy(accA.at[tslice], sbuf)

        @pl.loop(0, STILE, step=16)
        def _(i):
            d16 = dsm[pl.ds(t + i, 16)]
            for k in range(16):
                d = d16[k]
                sbuf[i + k, pl.ds(0, 16)] = sbuf[i + k, pl.ds(0, 16)] * d
                sbuf[i + k, pl.ds(16, 16)] = sbuf[i + k, pl.ds(16, 16)] * d

        pltpu.sync_copy(sbuf, accA.at[tslice])
        pltpu.sync_copy(sbuf, accB.at[tslice])

    plsc.subcore_barrier()

    edge_pass(accB, accA)
    plsc.subcore_barrier()

    pltpu.sync_copy(dis_hbm.at[nslice], dsm)
    b0 = bvec[pl.ds(0, 16)]
    b1 = bvec[pl.ds(16, 16)]

    @pl.loop(0, rps, step=STILE)
    def _(t):
        pltpu.sync_copy(accA.at[pl.ds(nbase + t, STILE)], sbuf)

        @pl.loop(0, STILE, step=16)
        def _(i):
            d16 = dsm[pl.ds(t + i, 16)]
            for k in range(16):
                d = d16[k]
                sbuf[i + k, pl.ds(0, 16)] = sbuf[i + k, pl.ds(0, 16)] * d + b0
                sbuf[i + k, pl.ds(16, 16)] = sbuf[i + k, pl.ds(16, 16)] * d + b1

        pltpu.sync_copy(sbuf, out_hbm.at[pl.ds(nbase + t, STILE),
                                         pl.ds(c * HALF, HALF)])


def _mm_body(x_ref, w_ref, p_ref, y_ref, dinv_ref, dis_ref):
    deg = 1.0 + p_ref[:, 0:1] + p_ref[:, 1:2]
    dis = lax.rsqrt(deg)
    dinv_ref[...] = 1.0 / deg
    dis_ref[...] = dis
    xw = lax.dot_general(x_ref[...], w_ref[...], (((1,), (1,)), ((), ())),
                         preferred_element_type=jnp.float32)
    y = dis * xw
    y_ref[0] = y[:, :HALF]
    y_ref[1] = y[:, HALF:]


@jax.jit
def kernel(x, edge_index, W, b):
    n, d_in = x.shape
    d_out = W.shape[0]
    e = edge_index.shape[1]

    np_ = _pad_to(n, NSUB * LANES)
    ep = _pad_to(e, NCORE * NSUB * CJ * LANES)
    erows = ep // LANES

    x = x.astype(jnp.float32)
    src = edge_index[0].astype(jnp.int32)
    dst = edge_index[1].astype(jnp.int32)
    pad = jnp.full((ep - e,), np_ - 1, jnp.int32)
    src2 = jnp.concatenate([src, pad]).reshape(erows, LANES)
    dst2 = jnp.concatenate([dst, pad]).reshape(erows, LANES)
    x_pad = jnp.pad(x, ((0, np_ - n), (0, 0)))

    mesh = plsc.VectorSubcoreMesh(core_axis_name="c", subcore_axis_name="s")
    f32 = jnp.float32
    sc_params = pltpu.CompilerParams(use_tc_tiling_on_sc=False)
    rps = np_ // NSUB

    deg_call = pl.kernel(
        functools.partial(_deg_body, np_, erows // (NCORE * NSUB)),
        out_type=jax.ShapeDtypeStruct((NCORE, np_), f32),
        mesh=mesh,
        scratch_types=[
            pltpu.VMEM((CJ, LANES), jnp.int32),
            pltpu.VMEM((CJ, LANES), jnp.int32),
            pltpu.VMEM((LANES,), f32),
            pltpu.VMEM((rps,), f32),
            pltpu.VMEM_SHARED((np_,), f32),
            pltpu.SemaphoreType.DMA,
            pltpu.SemaphoreType.DMA,
        ],
        compiler_params=sc_params,
    )
    hops_call = pl.kernel(
        functools.partial(_hops_body, np_, erows // NSUB),
        out_type=jax.ShapeDtypeStruct((np_, NCORE * HALF), f32),
        mesh=mesh,
        scratch_types=[
            pltpu.VMEM((CJ, 2, LANES), jnp.int32),
            pltpu.VMEM((CJ, 2, LANES), jnp.int32),
            pltpu.VMEM((CJ * LANES, HALF), f32),
            pltpu.VMEM((CJ * LANES, HALF), f32),
            pltpu.VMEM((STILE, HALF), f32),
            pltpu.VMEM((HALF,), f32),
            pltpu.VMEM((rps,), f32),
            pltpu.VMEM_SHARED((np_, HALF), f32),
            pltpu.VMEM_SHARED((np_, HALF), f32),
            pltpu.SemaphoreType.DMA,
            pltpu.SemaphoreType.DMA,
            pltpu.SemaphoreType.DMA,
            pltpu.SemaphoreType.DMA,
        ],
        compiler_params=sc_params,
    )

    blk = 512
    grid = (np_ // blk,)
    mm_call = pl.pallas_call(
        _mm_body,
        grid=grid,
        in_specs=[
            pl.BlockSpec((blk, d_in), lambda i: (i, 0)),
            pl.BlockSpec((d_out, d_in), lambda i: (0, 0)),
            pl.BlockSpec((blk, NCORE), lambda i: (i, 0)),
        ],
        out_specs=[
            pl.BlockSpec((NCORE, blk, HALF), lambda i: (0, i, 0)),
            pl.BlockSpec((blk, 1), lambda i: (i, 0)),
            pl.BlockSpec((blk, 1), lambda i: (i, 0)),
        ],
        out_shape=[
            jax.ShapeDtypeStruct((NCORE, np_, HALF), f32),
            jax.ShapeDtypeStruct((np_, 1), f32),
            jax.ShapeDtypeStruct((np_, 1), f32),
        ],
    )

    p = deg_call(dst2)
    y0, dinv, dis = mm_call(x_pad, W, p.T)
    b2 = b.astype(f32).reshape(NCORE, HALF)
    e2 = jnp.stack([src2, dst2], axis=1)
    out = hops_call(y0, e2, dinv.reshape(np_), dis.reshape(np_), b2)
    return out[:n]


# speedup vs baseline: 2.0635x; 1.0267x over previous
import functools

import jax
import jax.numpy as jnp
from jax import lax
from jax.experimental import pallas as pl
from jax.experimental.pallas import tpu as pltpu
from jax.experimental.pallas import tpu_sc as plsc

NSUB = 16
NCORE = 2
HALF = 32
LANES = 128
CJ = 8
STILE = 128


def _pad_to(n, m):
    return -(-n // m) * m


def _deg_body(np_, rows_per_w, dst_hbm, out_hbm, idx0, idx1, ones_v, fill_v,
              dacc, t0, t1):
    c = lax.axis_index("c")
    s = lax.axis_index("s")
    rps = np_ // NSUB

    @pl.loop(0, LANES, step=16)
    def _(i):
        ones_v[pl.ds(i, 16)] = jnp.full((16,), 1.0, jnp.float32)

    @pl.loop(0, rps, step=16)
    def _(i):
        fill_v[pl.ds(i, 16)] = jnp.zeros((16,), jnp.float32)

    nslice = pl.ds(s * rps, rps)
    pltpu.sync_copy(fill_v, dacc.at[nslice])
    plsc.subcore_barrier()

    base = (c * NSUB + s) * rows_per_w
    nch = rows_per_w // CJ

    def load_idx(k, idx):
        pltpu.sync_copy(dst_hbm.at[pl.ds(base + k * CJ, CJ)], idx)

    def fireS(idx, sem):
        for j in range(CJ):
            pltpu.async_copy(ones_v, dacc.at[idx.at[j]], sem, add=True)

    def drainS(idx, sem):
        for j in range(CJ):
            pltpu.make_async_copy(dst_hbm.at[pl.ds(0, 1)].at[0],
                                  idx.at[0].at[pl.ds(0, LANES)], sem).wait()

    load_idx(0, idx0)
    fireS(idx0, t0)
    load_idx(1, idx1)
    fireS(idx1, t1)

    @pl.loop(2, nch, step=2)
    def _(k):
        drainS(idx0, t0)
        load_idx(k, idx0)
        fireS(idx0, t0)
        drainS(idx1, t1)
        load_idx(k + 1, idx1)
        fireS(idx1, t1)

    drainS(idx0, t0)
    drainS(idx1, t1)
    plsc.subcore_barrier()
    pltpu.sync_copy(dacc.at[nslice], out_hbm.at[c].at[nslice])


def _hops_body(np_, rows_per_s, y_hbm, e_hbm, dinv_hbm, dis_hbm,
               b_hbm, out_hbm,
               eidx0, eidx1, eidx2, eidx3, row0, row1, sbuf, bvec,
               dsm, accA, accB, g0, g1, t0, t1, i0, i1, i2, i3):
    c = lax.axis_index("c")
    s = lax.axis_index("s")
    rps = np_ // NSUB
    nbase = s * rps
    nslice = pl.ds(nbase, rps)
    ebase = s * rows_per_s
    nch = rows_per_s // CJ

    eidx = (eidx0, eidx1, eidx2, eidx3)
    rows = (row0, row1)
    gsem = (g0, g1)
    tsem = (t0, t1)
    isem = (i0, i1, i2, i3)

    def fireG(src_ref, ei, row, sem):
        for j in range(CJ):
            pltpu.async_copy(src_ref.at[ei.at[j, 0]],
                             row.at[pl.ds(j * LANES, LANES)], sem)

    def drainG(row, sem):
        pltpu.make_async_copy(y_hbm.at[c].at[pl.ds(0, CJ * LANES)],
                              row, sem).wait()

    def fireS(ei, row, acc, sem):
        for j in range(CJ):
            pltpu.async_copy(row.at[pl.ds(j * LANES, LANES)],
                             acc.at[ei.at[j, 1]], sem, add=True)

    drainS = drainG

    def fire_idx(k, slot):
        pltpu.async_copy(e_hbm.at[pl.ds(ebase + k * CJ, CJ)],
                         eidx[slot], isem[slot])

    def iwait(slot):
        pltpu.make_async_copy(e_hbm.at[pl.ds(0, CJ)], eidx[slot],
                              isem[slot]).wait()

    def edge_pass(src_ref, acc):
        def block(j, has_prev, do_prefetch, do_next):
            rs = j % 2
            if has_prev:
                drainS(rows[1 - rs], tsem[1 - rs])
            if do_prefetch:
                fire_idx(j + 3, (j - 1) % 4)
            drainG(rows[rs], gsem[rs])
            if do_next:
                iwait((j + 1) % 4)
                fireG(src_ref, eidx[(j + 1) % 4], rows[1 - rs],
                      gsem[1 - rs])
            fireS(eidx[j % 4], rows[rs], acc, tsem[rs])

        for k in range(4):
            fire_idx(k, k)
        iwait(0)
        fireG(src_ref, eidx0, row0, g0)
        for j in range(4):
            block(j, j >= 1, j >= 1, True)

        @pl.loop(4, nch - 4, step=4)
        def _(k):
            for dj in range(4):
                jj = k + dj
                rs = dj % 2
                drainS(rows[1 - rs], tsem[1 - rs])
                fire_idx(jj + 3, (dj - 1) % 4)
                drainG(rows[rs], gsem[rs])
                iwait((dj + 1) % 4)
                fireG(src_ref, eidx[(dj + 1) % 4], rows[1 - rs],
                      gsem[1 - rs])
                fireS(eidx[dj % 4], rows[rs], acc, tsem[rs])

        for j in range(nch - 4, nch):
            block(j, True, j + 3 < nch, j + 1 < nch)
        drainS(rows[(nch - 1) % 2], tsem[(nch - 1) % 2])

    pltpu.sync_copy(y_hbm.at[c].at[nslice], accA.at[nslice])
    pltpu.sync_copy(b_hbm.at[c], bvec)
    plsc.subcore_barrier()
    edge_pass(y_hbm.at[c], accA)
    plsc.subcore_barrier()

    pltpu.sync_copy(dinv_hbm.at[nslice], dsm)

    @pl.loop(0, rps, step=STILE)
    def _(t):
        tslice = pl.ds(nbase + t, STILE)
        pltpu.sync_copy(accA.at[tslice], sbuf)

        @pl.loop(0, STILE, step=16)
        def _(i):
            d16 = dsm[pl.ds(t + i, 16)]
            for k in range(16):
                d = d16[k]
                sbuf[i + k, pl.ds(0, 16)] = sbuf[i + k, pl.ds(0, 16)] * d
                sbuf[i + k, pl.ds(16, 16)] = sbuf[i + k, pl.ds(16, 16)] * d

        pltpu.sync_copy(sbuf, accA.at[tslice])
        pltpu.sync_copy(sbuf, accB.at[tslice])

    plsc.subcore_barrier()

    edge_pass(accB, accA)
    plsc.subcore_barrier()

    pltpu.sync_copy(dis_hbm.at[nslice], dsm)
    b0 = bvec[pl.ds(0, 16)]
    b1 = bvec[pl.ds(16, 16)]

    @pl.loop(0, rps, step=STILE)
    def _(t):
        pltpu.sync_copy(accA.at[pl.ds(nbase + t, STILE)], sbuf)

        @pl.loop(0, STILE, step=16)
        def _(i):
            d16 = dsm[pl.ds(t + i, 16)]
            for k in range(16):
                d = d16[k]
                sbuf[i + k, pl.ds(0, 16)] = sbuf[i + k, pl.ds(0, 16)] * d + b0
                sbuf[i + k, pl.ds(16, 16)] = sbuf[i + k, pl.ds(16, 16)] * d + b1

        pltpu.sync_copy(sbuf, out_hbm.at[pl.ds(nbase + t, STILE),
                                         pl.ds(c * HALF, HALF)])


def _mm_body(x_ref, w_ref, p_ref, y_ref, dinv_ref, dis_ref):
    deg = 1.0 + p_ref[:, 0:1] + p_ref[:, 1:2]
    dis = lax.rsqrt(deg)
    dinv_ref[...] = 1.0 / deg
    dis_ref[...] = dis
    xw = lax.dot_general(x_ref[...], w_ref[...], (((1,), (1,)), ((), ())),
                         preferred_element_type=jnp.float32)
    y = dis * xw
    y_ref[0] = y[:, :HALF]
    y_ref[1] = y[:, HALF:]


@jax.jit
def kernel(x, edge_index, W, b):
    n, d_in = x.shape
    d_out = W.shape[0]
    e = edge_index.shape[1]

    np_ = _pad_to(n, NSUB * LANES)
    ep = _pad_to(e, NCORE * NSUB * CJ * LANES)
    erows = ep // LANES

    x = x.astype(jnp.float32)
    src = edge_index[0].astype(jnp.int32)
    dst = edge_index[1].astype(jnp.int32)
    pad = jnp.full((ep - e,), np_ - 1, jnp.int32)
    src2 = jnp.concatenate([src, pad]).reshape(erows, LANES)
    dst2 = jnp.concatenate([dst, pad]).reshape(erows, LANES)
    x_pad = jnp.pad(x, ((0, np_ - n), (0, 0)))

    mesh = plsc.VectorSubcoreMesh(core_axis_name="c", subcore_axis_name="s")
    f32 = jnp.float32
    sc_params = pltpu.CompilerParams(use_tc_tiling_on_sc=False)
    rps = np_ // NSUB

    deg_call = pl.kernel(
        functools.partial(_deg_body, np_, erows // (NCORE * NSUB)),
        out_type=jax.ShapeDtypeStruct((NCORE, np_), f32),
        mesh=mesh,
        scratch_types=[
            pltpu.VMEM((CJ, LANES), jnp.int32),
            pltpu.VMEM((CJ, LANES), jnp.int32),
            pltpu.VMEM((LANES,), f32),
            pltpu.VMEM((rps,), f32),
            pltpu.VMEM_SHARED((np_,), f32),
            pltpu.SemaphoreType.DMA,
            pltpu.SemaphoreType.DMA,
        ],
        compiler_params=sc_params,
    )
    hops_call = pl.kernel(
        functools.partial(_hops_body, np_, erows // NSUB),
        out_type=jax.ShapeDtypeStruct((np_, NCORE * HALF), f32),
        mesh=mesh,
        scratch_types=[
            pltpu.VMEM((CJ, 2, LANES), jnp.int32),
            pltpu.VMEM((CJ, 2, LANES), jnp.int32),
            pltpu.VMEM((CJ, 2, LANES), jnp.int32),
            pltpu.VMEM((CJ, 2, LANES), jnp.int32),
            pltpu.VMEM((CJ * LANES, HALF), f32),
            pltpu.VMEM((CJ * LANES, HALF), f32),
            pltpu.VMEM((STILE, HALF), f32),
            pltpu.VMEM((HALF,), f32),
            pltpu.VMEM((rps,), f32),
            pltpu.VMEM_SHARED((np_, HALF), f32),
            pltpu.VMEM_SHARED((np_, HALF), f32),
            pltpu.SemaphoreType.DMA,
            pltpu.SemaphoreType.DMA,
            pltpu.SemaphoreType.DMA,
            pltpu.SemaphoreType.DMA,
            pltpu.SemaphoreType.DMA,
            pltpu.SemaphoreType.DMA,
            pltpu.SemaphoreType.DMA,
            pltpu.SemaphoreType.DMA,
        ],
        compiler_params=sc_params,
    )

    blk = 512
    grid = (np_ // blk,)
    mm_call = pl.pallas_call(
        _mm_body,
        grid=grid,
        in_specs=[
            pl.BlockSpec((blk, d_in), lambda i: (i, 0)),
            pl.BlockSpec((d_out, d_in), lambda i: (0, 0)),
            pl.BlockSpec((blk, NCORE), lambda i: (i, 0)),
        ],
        out_specs=[
            pl.BlockSpec((NCORE, blk, HALF), lambda i: (0, i, 0)),
            pl.BlockSpec((blk, 1), lambda i: (i, 0)),
            pl.BlockSpec((blk, 1), lambda i: (i, 0)),
        ],
        out_shape=[
            jax.ShapeDtypeStruct((NCORE, np_, HALF), f32),
            jax.ShapeDtypeStruct((np_, 1), f32),
            jax.ShapeDtypeStruct((np_, 1), f32),
        ],
    )

    p = deg_call(dst2)
    y0, dinv, dis = mm_call(x_pad, W, p.T)
    b2 = b.astype(f32).reshape(NCORE, HALF)
    e2 = jnp.stack([src2, dst2], axis=1)
    out = hops_call(y0, e2, dinv.reshape(np_), dis.reshape(np_), b2)
    return out[:n]


# speedup vs baseline: 2.1096x; 1.0223x over previous
import functools

import jax
import jax.numpy as jnp
from jax import lax
from jax.experimental import pallas as pl
from jax.experimental.pallas import tpu as pltpu
from jax.experimental.pallas import tpu_sc as plsc

NSUB = 16
NCORE = 2
HALF = 32
LANES = 128
CJ = 8
STILE = 128


def _pad_to(n, m):
    return -(-n // m) * m


def _deg_body(np_, rows_per_w, dst_hbm, out_hbm, idx0, idx1, ones_v, fill_v,
              dacc, t0, t1):
    c = lax.axis_index("c")
    s = lax.axis_index("s")
    rps = np_ // NSUB

    @pl.loop(0, LANES, step=16)
    def _(i):
        ones_v[pl.ds(i, 16)] = jnp.full((16,), 1.0, jnp.float32)

    @pl.loop(0, rps, step=16)
    def _(i):
        fill_v[pl.ds(i, 16)] = jnp.zeros((16,), jnp.float32)

    nslice = pl.ds(s * rps, rps)
    pltpu.sync_copy(fill_v, dacc.at[nslice])
    plsc.subcore_barrier()

    base = (c * NSUB + s) * rows_per_w
    nch = rows_per_w // CJ

    def load_idx(k, idx):
        pltpu.sync_copy(dst_hbm.at[pl.ds(base + k * CJ, CJ)], idx)

    def fireS(idx, sem):
        for j in range(CJ):
            pltpu.async_copy(ones_v, dacc.at[idx.at[j]], sem, add=True)

    def drainS(idx, sem):
        for j in range(CJ):
            pltpu.make_async_copy(dst_hbm.at[pl.ds(0, 1)].at[0],
                                  idx.at[0].at[pl.ds(0, LANES)], sem).wait()

    load_idx(0, idx0)
    fireS(idx0, t0)
    load_idx(1, idx1)
    fireS(idx1, t1)

    @pl.loop(2, nch, step=2)
    def _(k):
        drainS(idx0, t0)
        load_idx(k, idx0)
        fireS(idx0, t0)
        drainS(idx1, t1)
        load_idx(k + 1, idx1)
        fireS(idx1, t1)

    drainS(idx0, t0)
    drainS(idx1, t1)
    plsc.subcore_barrier()
    pltpu.sync_copy(dacc.at[nslice], out_hbm.at[c].at[nslice])


def _hops_body(np_, rows_per_s, y_hbm, e_hbm, dinv_hbm, dis_hbm,
               b_hbm, out_hbm,
               eidx0, eidx1, eidx2, eidx3, row0, row1, sbuf, bvec,
               dsm, accA, accB, g0, g1, t0, t1, i0, i1, i2, i3):
    c = lax.axis_index("c")
    s = lax.axis_index("s")
    rps = np_ // NSUB
    nbase = s * rps
    nslice = pl.ds(nbase, rps)
    ebase = s * rows_per_s
    nch = rows_per_s // CJ

    eidx = (eidx0, eidx1, eidx2, eidx3)
    rows = (row0, row1)
    gsem = (g0, g1)
    tsem = (t0, t1)
    isem = (i0, i1, i2, i3)

    def fireG(src_ref, ei, row, sem):
        for j in range(CJ):
            pltpu.async_copy(src_ref.at[ei.at[j, 0]],
                             row.at[pl.ds(j * LANES, LANES)], sem)

    def drainG(row, sem):
        pltpu.make_async_copy(y_hbm.at[c].at[pl.ds(0, CJ * LANES)],
                              row, sem).wait()

    def fireS(ei, row, acc, sem):
        for j in range(CJ):
            pltpu.async_copy(row.at[pl.ds(j * LANES, LANES)],
                             acc.at[ei.at[j, 1]], sem, add=True)

    drainS = drainG

    def fire_idx(k, slot):
        pltpu.async_copy(e_hbm.at[pl.ds(ebase + k * CJ, CJ)],
                         eidx[slot], isem[slot])

    def iwait(slot):
        pltpu.make_async_copy(e_hbm.at[pl.ds(0, CJ)], eidx[slot],
                              isem[slot]).wait()

    def edge_pass(src_ref, acc):
        def block(j, has_prev, do_prefetch, do_next):
            rs = j % 2
            if has_prev:
                drainS(rows[1 - rs], tsem[1 - rs])
            if do_prefetch:
                fire_idx(j + 3, (j - 1) % 4)
            drainG(rows[rs], gsem[rs])
            if do_next:
                iwait((j + 1) % 4)
                fireG(src_ref, eidx[(j + 1) % 4], rows[1 - rs],
                      gsem[1 - rs])
            fireS(eidx[j % 4], rows[rs], acc, tsem[rs])

        for k in range(4):
            fire_idx(k, k)
        iwait(0)
        fireG(src_ref, eidx0, row0, g0)
        for j in range(4):
            block(j, j >= 1, j >= 1, True)

        @pl.loop(4, nch - 4, step=4)
        def _(k):
            for dj in range(4):
                jj = k + dj
                rs = dj % 2
                drainS(rows[1 - rs], tsem[1 - rs])
                fire_idx(jj + 3, (dj - 1) % 4)
                drainG(rows[rs], gsem[rs])
                iwait((dj + 1) % 4)
                fireG(src_ref, eidx[(dj + 1) % 4], rows[1 - rs],
                      gsem[1 - rs])
                fireS(eidx[dj % 4], rows[rs], acc, tsem[rs])

        for j in range(nch - 4, nch):
            block(j, True, j + 3 < nch, j + 1 < nch)
        drainS(rows[(nch - 1) % 2], tsem[(nch - 1) % 2])

    pltpu.sync_copy(y_hbm.at[c].at[nslice], accA.at[nslice])
    pltpu.sync_copy(b_hbm.at[c], bvec)
    plsc.subcore_barrier()
    edge_pass(y_hbm.at[c], accA)
    plsc.subcore_barrier()

    pltpu.sync_copy(dinv_hbm.at[nslice], dsm)

    @pl.loop(0, rps, step=STILE)
    def _(t):
        tslice = pl.ds(nbase + t, STILE)
        pltpu.sync_copy(accA.at[tslice], sbuf)

        @pl.loop(0, STILE, step=16)
        def _(i):
            d16 = dsm[pl.ds(t + i, 16)]
            for k in range(16):
                d = d16[k]
                sbuf[i + k, pl.ds(0, 16)] = sbuf[i + k, pl.ds(0, 16)] * d
                sbuf[i + k, pl.ds(16, 16)] = sbuf[i + k, pl.ds(16, 16)] * d

        pltpu.sync_copy(sbuf, accA.at[tslice])
        pltpu.sync_copy(sbuf, accB.at[tslice])

    plsc.subcore_barrier()

    edge_pass(accB, accA)
    plsc.subcore_barrier()

    pltpu.sync_copy(dis_hbm.at[nslice], dsm)
    b0 = bvec[pl.ds(0, 16)]
    b1 = bvec[pl.ds(16, 16)]

    @pl.loop(0, rps, step=STILE)
    def _(t):
        pltpu.sync_copy(accA.at[pl.ds(nbase + t, STILE)], sbuf)

        @pl.loop(0, STILE, step=16)
        def _(i):
            d16 = dsm[pl.ds(t + i, 16)]
            for k in range(16):
                d = d16[k]
                sbuf[i + k, pl.ds(0, 16)] = sbuf[i + k, pl.ds(0, 16)] * d + b0
                sbuf[i + k, pl.ds(16, 16)] = sbuf[i + k, pl.ds(16, 16)] * d + b1

        pltpu.sync_copy(sbuf, out_hbm.at[pl.ds(nbase + t, STILE),
                                         pl.ds(c * HALF, HALF)])


def _mm_body(x_ref, w_ref, p_ref, y_ref, dinv_ref, dis_ref):
    deg = 1.0 + p_ref[:, 0:1] + p_ref[:, 1:2]
    dis = lax.rsqrt(deg)
    dinv_ref[...] = 1.0 / deg
    dis_ref[...] = dis
    xw = lax.dot_general(x_ref[...], w_ref[...], (((1,), (1,)), ((), ())),
                         preferred_element_type=jnp.float32)
    y = dis * xw
    y_ref[0] = y[:, :HALF]
    y_ref[1] = y[:, HALF:]


@jax.jit
def kernel(x, edge_index, W, b):
    n, d_in = x.shape
    d_out = W.shape[0]
    e = edge_index.shape[1]

    np_ = _pad_to(n, NSUB * LANES)
    ep = _pad_to(e, NCORE * NSUB * CJ * LANES)
    erows = ep // LANES

    x = x.astype(jnp.float32)
    src = edge_index[0].astype(jnp.int32)
    dst = edge_index[1].astype(jnp.int32)
    pad = jnp.full((ep - e,), np_ - 1, jnp.int32)
    src2 = jnp.concatenate([src, pad]).reshape(erows, LANES)
    dst2 = jnp.concatenate([dst, pad]).reshape(erows, LANES)
    x_pad = jnp.pad(x, ((0, np_ - n), (0, 0)))

    mesh = plsc.VectorSubcoreMesh(core_axis_name="c", subcore_axis_name="s")
    f32 = jnp.float32
    sc_params = pltpu.CompilerParams(use_tc_tiling_on_sc=False)
    rps = np_ // NSUB

    deg_call = pl.kernel(
        functools.partial(_deg_body, np_, erows // (NCORE * NSUB)),
        out_type=jax.ShapeDtypeStruct((NCORE, np_), f32),
        mesh=mesh,
        scratch_types=[
            pltpu.VMEM((CJ, LANES), jnp.int32),
            pltpu.VMEM((CJ, LANES), jnp.int32),
            pltpu.VMEM((LANES,), f32),
            pltpu.VMEM((rps,), f32),
            pltpu.VMEM_SHARED((np_,), f32),
            pltpu.SemaphoreType.DMA,
            pltpu.SemaphoreType.DMA,
        ],
        compiler_params=sc_params,
    )
    hops_call = pl.kernel(
        functools.partial(_hops_body, np_, erows // NSUB),
        out_type=jax.ShapeDtypeStruct((np_, NCORE * HALF), f32),
        mesh=mesh,
        scratch_types=[
            pltpu.VMEM((CJ, 2, LANES), jnp.int32),
            pltpu.VMEM((CJ, 2, LANES), jnp.int32),
            pltpu.VMEM((CJ, 2, LANES), jnp.int32),
            pltpu.VMEM((CJ, 2, LANES), jnp.int32),
            pltpu.VMEM((CJ * LANES, HALF), f32),
            pltpu.VMEM((CJ * LANES, HALF), f32),
            pltpu.VMEM((STILE, HALF), f32),
            pltpu.VMEM((HALF,), f32),
            pltpu.VMEM((rps,), f32),
            pltpu.VMEM_SHARED((np_, HALF), f32),
            pltpu.VMEM_SHARED((np_, HALF), f32),
            pltpu.SemaphoreType.DMA,
            pltpu.SemaphoreType.DMA,
            pltpu.SemaphoreType.DMA,
            pltpu.SemaphoreType.DMA,
            pltpu.SemaphoreType.DMA,
            pltpu.SemaphoreType.DMA,
            pltpu.SemaphoreType.DMA,
            pltpu.SemaphoreType.DMA,
        ],
        compiler_params=sc_params,
    )

    blk = 2048
    grid = (np_ // blk,)
    mm_call = pl.pallas_call(
        _mm_body,
        grid=grid,
        in_specs=[
            pl.BlockSpec((blk, d_in), lambda i: (i, 0)),
            pl.BlockSpec((d_out, d_in), lambda i: (0, 0)),
            pl.BlockSpec((blk, NCORE), lambda i: (i, 0)),
        ],
        out_specs=[
            pl.BlockSpec((NCORE, blk, HALF), lambda i: (0, i, 0)),
            pl.BlockSpec((blk, 1), lambda i: (i, 0)),
            pl.BlockSpec((blk, 1), lambda i: (i, 0)),
        ],
        out_shape=[
            jax.ShapeDtypeStruct((NCORE, np_, HALF), f32),
            jax.ShapeDtypeStruct((np_, 1), f32),
            jax.ShapeDtypeStruct((np_, 1), f32),
        ],
    )

    p = deg_call(dst2)
    y0, dinv, dis = mm_call(x_pad, W, p.T)
    b2 = b.astype(f32).reshape(NCORE, HALF)
    e2 = jnp.stack([src2, dst2], axis=1)
    out = hops_call(y0, e2, dinv.reshape(np_), dis.reshape(np_), b2)
    return out[:n]


# speedup vs baseline: 2.1132x; 1.0017x over previous
import functools

import jax
import jax.numpy as jnp
from jax import lax
from jax.experimental import pallas as pl
from jax.experimental.pallas import tpu as pltpu
from jax.experimental.pallas import tpu_sc as plsc

NSUB = 16
NCORE = 2
HALF = 32
LANES = 128
CJ = 8
STILE = 128


def _pad_to(n, m):
    return -(-n // m) * m


def _deg_body(np_, rows_per_w, dst_hbm, out_hbm, idx0, idx1, ones_v, fill_v,
              dacc, t0, t1):
    c = lax.axis_index("c")
    s = lax.axis_index("s")
    rps = np_ // NSUB

    @pl.loop(0, LANES, step=16)
    def _(i):
        ones_v[pl.ds(i, 16)] = jnp.full((16,), 1.0, jnp.float32)

    @pl.loop(0, rps, step=16)
    def _(i):
        fill_v[pl.ds(i, 16)] = jnp.zeros((16,), jnp.float32)

    nslice = pl.ds(s * rps, rps)
    pltpu.sync_copy(fill_v, dacc.at[nslice])
    plsc.subcore_barrier()

    base = (c * NSUB + s) * rows_per_w
    nch = rows_per_w // CJ

    def load_idx(k, idx):
        pltpu.sync_copy(dst_hbm.at[pl.ds(base + k * CJ, CJ)], idx)

    def fireS(idx, sem):
        for j in range(CJ):
            pltpu.async_copy(ones_v, dacc.at[idx.at[j]], sem, add=True)

    def drainS(idx, sem):
        for j in range(CJ):
            pltpu.make_async_copy(dst_hbm.at[pl.ds(0, 1)].at[0],
                                  idx.at[0].at[pl.ds(0, LANES)], sem).wait()

    load_idx(0, idx0)
    fireS(idx0, t0)
    load_idx(1, idx1)
    fireS(idx1, t1)

    @pl.loop(2, nch, step=2)
    def _(k):
        drainS(idx0, t0)
        load_idx(k, idx0)
        fireS(idx0, t0)
        drainS(idx1, t1)
        load_idx(k + 1, idx1)
        fireS(idx1, t1)

    drainS(idx0, t0)
    drainS(idx1, t1)
    plsc.subcore_barrier()
    pltpu.sync_copy(dacc.at[nslice], out_hbm.at[c].at[nslice])


def _hops_body(np_, rows_per_s, y_hbm, e_hbm, dinv_hbm, dis_hbm,
               b_hbm, out_hbm,
               eidx0, eidx1, eidx2, eidx3, row0, row1, sbuf, bvec,
               dsm, accA, accB, g0, g1, t0, t1, i0, i1, i2, i3):
    c = lax.axis_index("c")
    s = lax.axis_index("s")
    rps = np_ // NSUB
    nbase = s * rps
    nslice = pl.ds(nbase, rps)
    ebase = s * rows_per_s
    nch = rows_per_s // CJ

    eidx = (eidx0, eidx1, eidx2, eidx3)
    rows = (row0, row1)
    gsem = (g0, g1)
    tsem = (t0, t1)
    isem = (i0, i1, i2, i3)

    def fireG(src_ref, ei, row, sem):
        for j in range(CJ):
            pltpu.async_copy(src_ref.at[ei.at[j, 0]],
                             row.at[pl.ds(j * LANES, LANES)], sem)

    def drainG(row, sem):
        pltpu.make_async_copy(y_hbm.at[c].at[pl.ds(0, CJ * LANES)],
                              row, sem).wait()

    def fireS(ei, row, acc, sem):
        for j in range(CJ):
            pltpu.async_copy(row.at[pl.ds(j * LANES, LANES)],
                             acc.at[ei.at[j, 1]], sem, add=True)

    drainS = drainG

    def fire_idx(k, slot):
        pltpu.async_copy(e_hbm.at[pl.ds(ebase + k * CJ, CJ)],
                         eidx[slot], isem[slot])

    def iwait(slot):
        pltpu.make_async_copy(e_hbm.at[pl.ds(0, CJ)], eidx[slot],
                              isem[slot]).wait()

    def edge_pass(src_ref, acc):
        def block(j, has_prev, do_prefetch, do_next):
            rs = j % 2
            if has_prev:
                drainS(rows[1 - rs], tsem[1 - rs])
            if do_prefetch:
                fire_idx(j + 3, (j - 1) % 4)
            drainG(rows[rs], gsem[rs])
            if do_next:
                iwait((j + 1) % 4)
                fireG(src_ref, eidx[(j + 1) % 4], rows[1 - rs],
                      gsem[1 - rs])
            fireS(eidx[j % 4], rows[rs], acc, tsem[rs])

        for k in range(4):
            fire_idx(k, k)
        iwait(0)
        fireG(src_ref, eidx0, row0, g0)
        for j in range(4):
            block(j, j >= 1, j >= 1, True)

        @pl.loop(4, nch - 4, step=4)
        def _(k):
            for dj in range(4):
                jj = k + dj
                rs = dj % 2
                drainS(rows[1 - rs], tsem[1 - rs])
                fire_idx(jj + 3, (dj - 1) % 4)
                drainG(rows[rs], gsem[rs])
                iwait((dj + 1) % 4)
                fireG(src_ref, eidx[(dj + 1) % 4], rows[1 - rs],
                      gsem[1 - rs])
                fireS(eidx[dj % 4], rows[rs], acc, tsem[rs])

        for j in range(nch - 4, nch):
            block(j, True, j + 3 < nch, j + 1 < nch)
        drainS(rows[(nch - 1) % 2], tsem[(nch - 1) % 2])

    pltpu.sync_copy(y_hbm.at[c].at[nslice], accA.at[nslice])
    pltpu.sync_copy(b_hbm.at[c], bvec)
    plsc.subcore_barrier()
    edge_pass(y_hbm.at[c], accA)
    plsc.subcore_barrier()

    pltpu.sync_copy(dinv_hbm.at[nslice], dsm)

    @pl.loop(0, rps, step=STILE)
    def _(t):
        tslice = pl.ds(nbase + t, STILE)
        pltpu.sync_copy(accA.at[tslice], sbuf)

        @pl.loop(0, STILE, step=16)
        def _(i):
            d16 = dsm[pl.ds(t + i, 16)]
            for k in range(16):
                d = d16[k]
                sbuf[i + k, pl.ds(0, 16)] = sbuf[i + k, pl.ds(0, 16)] * d
                sbuf[i + k, pl.ds(16, 16)] = sbuf[i + k, pl.ds(16, 16)] * d

        pltpu.sync_copy(sbuf, accA.at[tslice])
        pltpu.sync_copy(sbuf, accB.at[tslice])

    plsc.subcore_barrier()

    edge_pass(accB, accA)
    plsc.subcore_barrier()

    pltpu.sync_copy(dis_hbm.at[nslice], dsm)
    b0 = bvec[pl.ds(0, 16)]
    b1 = bvec[pl.ds(16, 16)]

    @pl.loop(0, rps, step=STILE)
    def _(t):
        pltpu.sync_copy(accA.at[pl.ds(nbase + t, STILE)], sbuf)

        @pl.loop(0, STILE, step=16)
        def _(i):
            d16 = dsm[pl.ds(t + i, 16)]
            for k in range(16):
                d = d16[k]
                sbuf[i + k, pl.ds(0, 16)] = sbuf[i + k, pl.ds(0, 16)] * d + b0
                sbuf[i + k, pl.ds(16, 16)] = sbuf[i + k, pl.ds(16, 16)] * d + b1

        pltpu.sync_copy(sbuf, out_hbm.at[pl.ds(nbase + t, STILE),
                                         pl.ds(c * HALF, HALF)])


def _mm_body(x_ref, w_ref, p_ref, y_ref, dinv_ref, dis_ref):
    deg = 1.0 + p_ref[:, 0:1] + p_ref[:, 1:2]
    dis = lax.rsqrt(deg)
    dinv_ref[...] = 1.0 / deg
    dis_ref[...] = dis
    xw = lax.dot_general(x_ref[...], w_ref[...], (((1,), (1,)), ((), ())),
                         preferred_element_type=jnp.float32)
    y = dis * xw
    y_ref[0] = y[:, :HALF]
    y_ref[1] = y[:, HALF:]


@jax.jit
def kernel(x, edge_index, W, b):
    n, d_in = x.shape
    d_out = W.shape[0]
    e = edge_index.shape[1]

    np_ = _pad_to(n, NSUB * LANES)
    ep = _pad_to(e, NCORE * NSUB * CJ * LANES)
    erows = ep // LANES

    x = x.astype(jnp.float32)
    src = edge_index[0].astype(jnp.int32)
    dst = edge_index[1].astype(jnp.int32)
    pad = jnp.full((ep - e,), np_ - 1, jnp.int32)
    src2 = jnp.concatenate([src, pad]).reshape(erows, LANES)
    dst2 = jnp.concatenate([dst, pad]).reshape(erows, LANES)
    x_pad = jnp.pad(x, ((0, np_ - n), (0, 0)))

    mesh = plsc.VectorSubcoreMesh(core_axis_name="c", subcore_axis_name="s")
    f32 = jnp.float32
    sc_params = pltpu.CompilerParams(use_tc_tiling_on_sc=False)
    rps = np_ // NSUB

    deg_call = pl.kernel(
        functools.partial(_deg_body, np_, erows // (NCORE * NSUB)),
        out_type=jax.ShapeDtypeStruct((NCORE, np_), f32),
        mesh=mesh,
        scratch_types=[
            pltpu.VMEM((CJ, LANES), jnp.int32),
            pltpu.VMEM((CJ, LANES), jnp.int32),
            pltpu.VMEM((LANES,), f32),
            pltpu.VMEM((rps,), f32),
            pltpu.VMEM_SHARED((np_,), f32),
            pltpu.SemaphoreType.DMA,
            pltpu.SemaphoreType.DMA,
        ],
        compiler_params=sc_params,
    )
    hops_call = pl.kernel(
        functools.partial(_hops_body, np_, erows // NSUB),
        out_type=jax.ShapeDtypeStruct((np_, NCORE * HALF), f32),
        mesh=mesh,
        scratch_types=[
            pltpu.VMEM((CJ, 2, LANES), jnp.int32),
            pltpu.VMEM((CJ, 2, LANES), jnp.int32),
            pltpu.VMEM((CJ, 2, LANES), jnp.int32),
            pltpu.VMEM((CJ, 2, LANES), jnp.int32),
            pltpu.VMEM((CJ * LANES, HALF), f32),
            pltpu.VMEM((CJ * LANES, HALF), f32),
            pltpu.VMEM((STILE, HALF), f32),
            pltpu.VMEM((HALF,), f32),
            pltpu.VMEM((rps,), f32),
            pltpu.VMEM_SHARED((np_, HALF), f32),
            pltpu.VMEM_SHARED((np_, HALF), f32),
            pltpu.SemaphoreType.DMA,
            pltpu.SemaphoreType.DMA,
            pltpu.SemaphoreType.DMA,
            pltpu.SemaphoreType.DMA,
            pltpu.SemaphoreType.DMA,
            pltpu.SemaphoreType.DMA,
            pltpu.SemaphoreType.DMA,
            pltpu.SemaphoreType.DMA,
        ],
        compiler_params=sc_params,
    )

    blk = 10240 if np_ == 10240 else 2048
    grid = (np_ // blk,)
    mm_call = pl.pallas_call(
        _mm_body,
        grid=grid,
        in_specs=[
            pl.BlockSpec((blk, d_in), lambda i: (i, 0)),
            pl.BlockSpec((d_out, d_in), lambda i: (0, 0)),
            pl.BlockSpec((blk, NCORE), lambda i: (i, 0)),
        ],
        out_specs=[
            pl.BlockSpec((NCORE, blk, HALF), lambda i: (0, i, 0)),
            pl.BlockSpec((blk, 1), lambda i: (i, 0)),
            pl.BlockSpec((blk, 1), lambda i: (i, 0)),
        ],
        out_shape=[
            jax.ShapeDtypeStruct((NCORE, np_, HALF), f32),
            jax.ShapeDtypeStruct((np_, 1), f32),
            jax.ShapeDtypeStruct((np_, 1), f32),
        ],
    )

    p = deg_call(dst2)
    y0, dinv, dis = mm_call(x_pad, W, p.T)
    b2 = b.astype(f32).reshape(NCORE, HALF)
    e2 = jnp.stack([src2, dst2], axis=1)
    out = hops_call(y0, e2, dinv.reshape(np_), dis.reshape(np_), b2)
    return out[:n]
